# baseline (speedup 1.0000x reference)
import functools

import jax
import jax.numpy as jnp
from jax import lax
from jax.experimental import pallas as pl
from jax.experimental.pallas import tpu as pltpu
from jax.experimental.pallas import tpu_sc as plsc

_NUM_CLUSTERS = 8192
_DIM = 1024
_BN = 512
_BIG = 2**30


_BM = 128


def _dist_body(f_ref, c_ref, cn_ref, dist_ref, ids_ref):
    f = f_ref[...]
    acc = None
    for kk in range(8):
        f_k = f_ref[:, pl.ds(kk * (_DIM // 8), _DIM // 8)]
        sq = f_k * f_k
        acc = sq if acc is None else acc + sq
    fn = jnp.sum(acc, axis=1, keepdims=True)

    c = c_ref[...]
    p = lax.dot_general(f, c, (((1,), (1,)), ((), ())),
                        preferred_element_type=jnp.float32)
    dist = (fn - 2.0 * p) + cn_ref[...]
    dist_ref[...] = dist

    row_min = jnp.min(dist, axis=1, keepdims=True)
    col_ids = lax.broadcasted_iota(jnp.int32, dist.shape, 1)
    cand = jnp.where(dist == row_min, col_ids, _BIG)
    ids_ref[...] = jnp.min(cand, axis=1, keepdims=True)


def _distances_and_ids(flat, centers, cnorm):
    m = flat.shape[0]
    n_blocks = m // _BM
    dist, ids = pl.pallas_call(
        _dist_body,
        grid=(n_blocks,),
        in_specs=[
            pl.BlockSpec((_BM, _DIM), lambda i: (i, 0)),
            pl.BlockSpec((_NUM_CLUSTERS, _DIM), lambda i: (0, 0)),
            pl.BlockSpec((1, _NUM_CLUSTERS), lambda i: (0, 0)),
        ],
        out_specs=[
            pl.BlockSpec((_BM, _NUM_CLUSTERS), lambda i: (i, 0)),
            pl.BlockSpec((_BM, 1), lambda i: (i, 0)),
        ],
        out_shape=[
            jax.ShapeDtypeStruct((m, _NUM_CLUSTERS), jnp.float32),
            jax.ShapeDtypeStruct((m, 1), jnp.int32),
        ],
    )(flat, centers, cnorm)
    return dist, ids


def _sc_gather(centers, ids):
    b = ids.shape[0]
    n_workers = 32
    chunk = 64
    per_w = b // n_workers
    n_chunks = per_w // chunk
    mesh = plsc.VectorSubcoreMesh(core_axis_name="c", subcore_axis_name="s")

    @functools.partial(
        pl.kernel, mesh=mesh,
        out_type=jax.ShapeDtypeStruct((b, _DIM), jnp.float32),
        scratch_types=[
            pltpu.VMEM((chunk,), jnp.int32),
            pltpu.VMEM((chunk, _DIM), jnp.float32),
            pltpu.SemaphoreType.DMA,
        ],
    )
    def k(table_hbm, idx_hbm, out_hbm, idx_v, rows_v, sem):
        wid = lax.axis_index("s") * 2 + lax.axis_index("c")
        for i in range(n_chunks):
            base = wid * per_w + i * chunk
            pltpu.sync_copy(idx_hbm.at[pl.ds(base, chunk)], idx_v)
            pltpu.async_copy(table_hbm.at[idx_v], rows_v, sem).wait()
            pltpu.sync_copy(rows_v, out_hbm.at[pl.ds(base, chunk), :])

    return k(centers, ids)


def kernel(features, centers):
    batch, seq, dim = features.shape
    flat = features.reshape(-1, dim)
    cnorm = jnp.sum(centers ** 2, axis=1)[None, :]
    dist, ids2d = _distances_and_ids(flat, centers, cnorm)
    ids = ids2d.reshape(-1)
    quantized = _sc_gather(centers, ids)
    return (quantized.reshape(batch, seq, dim),
            ids.reshape(batch, seq),
            dist)


# speedup vs baseline: 1.5515x; 1.5515x over previous
import functools

import jax
import jax.numpy as jnp
from jax import lax
from jax.experimental import pallas as pl
from jax.experimental.pallas import tpu as pltpu
from jax.experimental.pallas import tpu_sc as plsc

_NUM_CLUSTERS = 8192
_DIM = 1024
_BN = 512
_BIG = 2**30


def _dist_body(f_ref, c_ref, cn_ref, dist_ref, ids_ref,
               best_ref, bidx_ref, fn_ref):
    j = pl.program_id(0)

    @pl.when(j == 0)
    def _():
        acc = None
        for kk in range(8):
            f_k = f_ref[:, pl.ds(kk * (_DIM // 8), _DIM // 8)]
            sq = f_k * f_k
            acc = sq if acc is None else acc + sq
        fn_ref[...] = jnp.sum(acc, axis=1, keepdims=True)

    f = f_ref[...]
    c = c_ref[...]
    p = lax.dot_general(f, c, (((1,), (1,)), ((), ())),
                        preferred_element_type=jnp.float32)
    dist = (fn_ref[...] - 2.0 * p) + cn_ref[...]
    dist_ref[...] = dist

    row_min = jnp.min(dist, axis=1, keepdims=True)
    col_ids = lax.broadcasted_iota(jnp.int32, dist.shape, 1)
    cand = jnp.where(dist == row_min, col_ids, _BIG)
    row_arg = jnp.min(cand, axis=1, keepdims=True) + j * _BN

    @pl.when(j == 0)
    def _():
        best_ref[...] = row_min
        bidx_ref[...] = row_arg

    @pl.when(j > 0)
    def _():
        upd = row_min < best_ref[...]
        best_ref[...] = jnp.where(upd, row_min, best_ref[...])
        bidx_ref[...] = jnp.where(upd, row_arg, bidx_ref[...])

    @pl.when(j == pl.num_programs(0) - 1)
    def _():
        ids_ref[...] = bidx_ref[...]


def _distances_and_ids(flat, centers, cnorm):
    m = flat.shape[0]
    n_blocks = _NUM_CLUSTERS // _BN
    dist, ids = pl.pallas_call(
        _dist_body,
        grid=(n_blocks,),
        in_specs=[
            pl.BlockSpec((m, _DIM), lambda j: (0, 0)),
            pl.BlockSpec((_BN, _DIM), lambda j: (j, 0)),
            pl.BlockSpec((1, _BN), lambda j: (0, j)),
        ],
        out_specs=[
            pl.BlockSpec((m, _BN), lambda j: (0, j)),
            pl.BlockSpec((m, 1), lambda j: (0, 0)),
        ],
        out_shape=[
            jax.ShapeDtypeStruct((m, _NUM_CLUSTERS), jnp.float32),
            jax.ShapeDtypeStruct((m, 1), jnp.int32),
        ],
        scratch_shapes=[
            pltpu.VMEM((m, 1), jnp.float32),
            pltpu.VMEM((m, 1), jnp.int32),
            pltpu.VMEM((m, 1), jnp.float32),
        ],
    )(flat, centers, cnorm)
    return dist, ids


def _sc_gather(centers, ids):
    b = ids.shape[0]
    n_workers = 32
    chunk = 64
    per_w = b // n_workers
    n_chunks = per_w // chunk
    mesh = plsc.VectorSubcoreMesh(core_axis_name="c", subcore_axis_name="s")

    @functools.partial(
        pl.kernel, mesh=mesh,
        out_type=jax.ShapeDtypeStruct((b, _DIM), jnp.float32),
        scratch_types=[
            pltpu.VMEM((chunk,), jnp.int32),
            pltpu.VMEM((chunk, _DIM), jnp.float32),
            pltpu.SemaphoreType.DMA,
        ],
    )
    def k(table_hbm, idx_hbm, out_hbm, idx_v, rows_v, sem):
        wid = lax.axis_index("s") * 2 + lax.axis_index("c")
        for i in range(n_chunks):
            base = wid * per_w + i * chunk
            pltpu.sync_copy(idx_hbm.at[pl.ds(base, chunk)], idx_v)
            pltpu.async_copy(table_hbm.at[idx_v], rows_v, sem).wait()
            pltpu.sync_copy(rows_v, out_hbm.at[pl.ds(base, chunk), :])

    return k(centers, ids)


def kernel(features, centers):
    batch, seq, dim = features.shape
    flat = features.reshape(-1, dim)
    cnorm = jnp.sum(centers ** 2, axis=1)[None, :]
    dist, ids2d = _distances_and_ids(flat, centers, cnorm)
    ids = ids2d.reshape(-1)
    quantized = _sc_gather(centers, ids)
    return (quantized.reshape(batch, seq, dim),
            ids.reshape(batch, seq),
            dist)
